# baseline (speedup 1.0000x reference)
import functools

import jax
import jax.numpy as jnp
import numpy as np
from jax import lax
from jax.experimental import pallas as pl
from jax.experimental.pallas import tpu as pltpu
from jax.experimental.pallas import tpu_sc as plsc

_M = 1000000
_B = 4096
_D = 6
_BB = 512
_NW = 32
_RANGE = _M // _NW
_TBL = _RANGE + 110
_CAP = _B + 128
_E = _CAP * _D
_EROW = _E // 128


def _stain_body(t_ref, m_ref, o_ref):
    t = t_ref[...]
    m = m_ref[...]
    denom = jnp.sum(m, axis=2) + 1e-6
    mean = jnp.sum(t * m, axis=2) / denom
    var = jnp.sum(((t - mean[:, :, None]) ** 2) * m, axis=2) / denom
    std = jnp.sqrt(var + 1e-6)
    o_ref[...] = jnp.concatenate(
        [mean[:, 0:1], std[:, 0:1], mean[:, 1:2], std[:, 1:2],
         mean[:, 2:3], std[:, 2:3]], axis=1)


def _stain_extract(target, mask):
    B, C, P = target.shape
    return pl.pallas_call(
        _stain_body,
        grid=(B // _BB,),
        in_specs=[
            pl.BlockSpec((_BB, C, P), lambda i: (i, 0, 0)),
            pl.BlockSpec((_BB, 1, P), lambda i: (i, 0, 0)),
        ],
        out_specs=pl.BlockSpec((_BB, _D), lambda i: (i, 0)),
        out_shape=jax.ShapeDtypeStruct((B, _D), jnp.float32),
    )(target, mask)


def _sparse_body(mem_h, keys_h, hit_h, sm_h, rc_h, out_h,
                 keysv, hitv, table, ilist, klist, ilist2, klist2,
                 gbuf, idx2d, rcv, wbuf, sem):
    c = lax.axis_index("c")
    s = lax.axis_index("s")
    wid = s * 2 + c
    base = wid * _RANGE
    lanes = lax.iota(jnp.int32, 16)
    dump0 = _B + wid * 128

    pltpu.async_copy(keys_h, keysv, sem)
    pltpu.async_copy(hit_h, hitv, sem)
    pltpu.async_copy(rc_h, rcv, sem)
    pltpu.make_async_copy(keys_h, keysv, sem).wait()
    pltpu.make_async_copy(hit_h, hitv, sem).wait()
    pltpu.make_async_copy(rc_h, rcv, sem).wait()
    _C_ROW = [rcv[pl.ds(k * 16, 16)] for k in range(_D)]
    _C_COL = [rcv[pl.ds(96 + k * 16, 16)] for k in range(_D)]


    def _scan(r, cnt):
        for l in range(8):
            kv = keysv[pl.ds(r * 128 + l * 16, 16)]
            rel = kv - base
            owned = (rel >= 0) & (rel < _RANGE)
            iv = r * 128 + l * 16 + lanes
            plsc.store_compressed(ilist.at[pl.ds(cnt, 16)], iv, mask=owned)
            plsc.store_compressed(klist.at[pl.ds(cnt, 16)], kv, mask=owned)
            pc = plsc.all_reduce_population_count(owned)
            cnt = cnt + jnp.max(pc)
        return cnt
    cnt = lax.fori_loop(0, _B // 128, _scan, jnp.int32(0))
    ilist[pl.ds(cnt, 16)] = jnp.full((16,), dump0, jnp.int32) + lanes
    klist[pl.ds(cnt, 16)] = jnp.full((16,), base, jnp.int32)
    nv = (cnt + 15) >> 4

    def _build(v, carry):
        iv = ilist[pl.ds(v * 16, 16)]
        valid = iv < _B
        rel = klist[pl.ds(v * 16, 16)] - base
        plsc.store_scatter(table, [rel], iv, mask=valid)
        return carry
    lax.fori_loop(0, nv, _build, 0)

    def _fix(_):
        def _sweep(v, bad):
            iv = ilist[pl.ds(v * 16, 16)]
            valid = iv < _B
            rel = klist[pl.ds(v * 16, 16)] - base
            w = plsc.load_gather(table, [rel], mask=valid)
            isbad = valid & (iv > w)
            plsc.store_scatter(table, [rel], iv, mask=isbad)
            nb = plsc.all_reduce_population_count(isbad)
            return bad + jnp.max(nb)
        return lax.fori_loop(0, nv, _sweep, jnp.int32(0))
    lax.while_loop(lambda bad: bad > 0, _fix, jnp.int32(1))

    def _part(v, carry):
        cnth, cntm = carry
        iv = ilist[pl.ds(v * 16, 16)]
        valid = iv < _B
        kv = klist[pl.ds(v * 16, 16)]
        rel = kv - base
        w = plsc.load_gather(table, [rel], mask=valid)
        hw = plsc.load_gather(hitv, [w], mask=valid)
        hb = valid & (hw > 0)
        mb = valid & jnp.logical_not(hb)
        plsc.store_compressed(ilist2.at[pl.ds(cnth, 16)], iv, mask=hb)
        plsc.store_compressed(klist2.at[pl.ds(cnth, 16)], kv, mask=hb)
        ph = plsc.all_reduce_population_count(hb)
        plsc.store_compressed(ilist.at[pl.ds(cntm, 16)], iv, mask=mb)
        plsc.store_compressed(klist.at[pl.ds(cntm, 16)], kv, mask=mb)
        pm = plsc.all_reduce_population_count(mb)
        return (cnth + jnp.max(ph), cntm + jnp.max(pm))
    cnth, cntm = lax.fori_loop(0, nv, _part, (jnp.int32(0), jnp.int32(0)))
    ilist2[pl.ds(cnth, 16)] = jnp.full((16,), dump0, jnp.int32) + lanes
    klist2[pl.ds(cnth, 16)] = jnp.full((16,), base, jnp.int32)
    def _scrub(t, carry):
        ilist[pl.ds(cntm + t * 16, 16)] = (
            jnp.full((16,), dump0, jnp.int32) + t * 16 + lanes)
        klist[pl.ds(cntm + t * 16, 16)] = jnp.full((16,), base, jnp.int32)
        return carry
    lax.fori_loop(0, 2, _scrub, 0)

    nvh = (cnth + 15) >> 4
    nch = (cnth * _D + 127) >> 7
    nvm = (cntm + 15) >> 4
    ncm = (cntm * _D + 127) >> 7

    def _aidx(v, carry):
        for k in range(_D):
            kk = plsc.load_gather(klist2, [v * 16 + _C_ROW[k]])
            pos = v * 96 + k * 16
            idx2d[pos >> 7, pl.ds(pos & 127, 16)] = kk * _D + _C_COL[k]
        return carry
    lax.fori_loop(0, nvh, _aidx, 0)
    def _atail(t, carry):
        pos = nvh * 96 + t * 16
        idx2d[pos >> 7, pl.ds(pos & 127, 16)] = jnp.full((16,), base * _D, jnp.int32)
        return carry
    lax.fori_loop(0, 8, _atail, 0)

    def _afire(j, carry):
        pltpu.async_copy(mem_h.at[idx2d.at[j]],
                         gbuf.at[pl.ds(j * 128, 128)], sem)
        return carry
    lax.fori_loop(0, nch, _afire, 0)
    def _adrain(j, carry):
        pltpu.make_async_copy(mem_h.at[idx2d.at[j]],
                              gbuf.at[pl.ds(j * 128, 128)], sem).wait()
        return carry
    lax.fori_loop(0, nch, _adrain, 0)

    def _aodx(v, carry):
        for k in range(_D):
            ii = plsc.load_gather(ilist2, [v * 16 + _C_ROW[k]])
            pos = v * 96 + k * 16
            idx2d[pos >> 7, pl.ds(pos & 127, 16)] = ii * _D + _C_COL[k]
        return carry
    lax.fori_loop(0, nvh, _aodx, 0)
    def _aotail(t, carry):
        pos = nvh * 96 + t * 16
        idx2d[pos >> 7, pl.ds(pos & 127, 16)] = (
            jnp.full((16,), dump0 * _D + t * 16, jnp.int32) + lanes)
        return carry
    lax.fori_loop(0, 8, _aotail, 0)

    def _aofire(j, carry):
        pltpu.async_copy(gbuf.at[pl.ds(j * 128, 128)],
                         out_h.at[idx2d.at[j]], sem)
        return carry
    lax.fori_loop(0, nch, _aofire, 0)
    def _aodrain(j, carry):
        pltpu.make_async_copy(gbuf.at[pl.ds(j * 128, 128)],
                              out_h.at[idx2d.at[j]], sem).wait()
        return carry
    lax.fori_loop(0, nch, _aodrain, 0)

    def _bidx(v, carry):
        iv = ilist[pl.ds(v * 16, 16)]
        valid = iv < _B
        rel = klist[pl.ds(v * 16, 16)] - base
        w = plsc.load_gather(table, [rel], mask=valid)
        wbuf[...] = jnp.where(valid, w, 0) * _D
        for k in range(_D):
            w6 = plsc.load_gather(wbuf, [_C_ROW[k]])
            pos = v * 96 + k * 16
            idx2d[pos >> 7, pl.ds(pos & 127, 16)] = w6 + _C_COL[k]
        return carry
    lax.fori_loop(0, nvm, _bidx, 0)
    def _btail(t, carry):
        pos = nvm * 96 + t * 16
        idx2d[pos >> 7, pl.ds(pos & 127, 16)] = jnp.zeros((16,), jnp.int32)
        return carry
    lax.fori_loop(0, 8, _btail, 0)

    def _bfire(j, carry):
        pltpu.async_copy(sm_h.at[idx2d.at[j]],
                         gbuf.at[pl.ds(j * 128, 128)], sem)
        return carry
    lax.fori_loop(0, ncm, _bfire, 0)
    def _bdrain(j, carry):
        pltpu.make_async_copy(sm_h.at[idx2d.at[j]],
                              gbuf.at[pl.ds(j * 128, 128)], sem).wait()
        return carry
    lax.fori_loop(0, ncm, _bdrain, 0)

    def _bodx(v, carry):
        for k in range(_D):
            ii = plsc.load_gather(ilist, [v * 16 + _C_ROW[k]])
            pos = v * 96 + k * 16
            idx2d[pos >> 7, pl.ds(pos & 127, 16)] = ii * _D + _C_COL[k]
        return carry
    lax.fori_loop(0, nvm, _bodx, 0)
    def _botail(t, carry):
        pos = nvm * 96 + t * 16
        idx2d[pos >> 7, pl.ds(pos & 127, 16)] = (
            jnp.full((16,), dump0 * _D + t * 16, jnp.int32) + lanes)
        return carry
    lax.fori_loop(0, 8, _botail, 0)

    def _bofire(j, carry):
        pltpu.async_copy(gbuf.at[pl.ds(j * 128, 128)],
                         out_h.at[idx2d.at[j]], sem)
        return carry
    lax.fori_loop(0, ncm, _bofire, 0)
    def _bodrain(j, carry):
        pltpu.make_async_copy(gbuf.at[pl.ds(j * 128, 128)],
                              out_h.at[idx2d.at[j]], sem).wait()
        return carry
    lax.fori_loop(0, ncm, _bodrain, 0)


_sparse = functools.partial(
    pl.kernel,
    out_type=jax.ShapeDtypeStruct(((_B + _NW * 128) * _D,), jnp.float32),
    mesh=plsc.VectorSubcoreMesh(core_axis_name="c", subcore_axis_name="s"),
    scratch_types=[
        pltpu.VMEM((_B,), jnp.int32),
        pltpu.VMEM((_B,), jnp.int32),
        pltpu.VMEM((_TBL,), jnp.int32),
        pltpu.VMEM((_CAP,), jnp.int32),
        pltpu.VMEM((_CAP,), jnp.int32),
        pltpu.VMEM((_CAP,), jnp.int32),
        pltpu.VMEM((_CAP,), jnp.int32),
        pltpu.VMEM((_E,), jnp.float32),
        pltpu.VMEM((_EROW, 128), jnp.int32),
        pltpu.VMEM((192,), jnp.int32),
        pltpu.VMEM((16,), jnp.int32),
        pltpu.SemaphoreType.DMA,
    ],
    compiler_params=pltpu.CompilerParams(needs_layout_passes=False),
)(_sparse_body)


def kernel(mem, target, mask, keys, hit_flags):
    sm_miss = _stain_extract(target, mask)
    keys32 = keys.astype(jnp.int32)
    hit32 = hit_flags.astype(jnp.int32)
    ele = np.arange(96)
    rc = np.concatenate([ele // _D, ele % _D]).astype(np.int32)
    out_flat = _sparse(mem.reshape(-1), keys32, hit32, sm_miss.reshape(-1),
                       jnp.asarray(rc))
    return out_flat[:_B * _D].reshape(_B, _D)


# speedup vs baseline: 1.4582x; 1.4582x over previous
import functools

import jax
import jax.numpy as jnp
import numpy as np
from jax import lax
from jax.experimental import pallas as pl
from jax.experimental.pallas import tpu as pltpu
from jax.experimental.pallas import tpu_sc as plsc

_M = 1000000
_B = 4096
_D = 6
_BB = 512
_NW = 32
_RANGE = _M // _NW
_TBL = _RANGE + 110
_CAP = _B + 128
_E = _CAP * _D
_EROW = _E // 128


def _stain_body(t_ref, m_ref, o_ref):
    t = t_ref[...]
    m = m_ref[...]
    denom = jnp.sum(m, axis=2) + 1e-6
    mean = jnp.sum(t * m, axis=2) / denom
    var = jnp.sum(((t - mean[:, :, None]) ** 2) * m, axis=2) / denom
    std = jnp.sqrt(var + 1e-6)
    o_ref[...] = jnp.concatenate(
        [mean[:, 0:1], std[:, 0:1], mean[:, 1:2], std[:, 1:2],
         mean[:, 2:3], std[:, 2:3]], axis=1)


def _stain_extract(target, mask):
    B, C, P = target.shape
    return pl.pallas_call(
        _stain_body,
        grid=(B // _BB,),
        in_specs=[
            pl.BlockSpec((_BB, C, P), lambda i: (i, 0, 0)),
            pl.BlockSpec((_BB, 1, P), lambda i: (i, 0, 0)),
        ],
        out_specs=pl.BlockSpec((_BB, _D), lambda i: (i, 0)),
        out_shape=jax.ShapeDtypeStruct((B, _D), jnp.float32),
    )(target, mask)


def _sparse_body(mem_h, keys_h, hit_h, sm_h, out_h,
                 keysv, hitv, table, ilist, klist, ilist2, klist2, sem):
    c = lax.axis_index("c")
    s = lax.axis_index("s")
    wid = s * 2 + c
    base = wid * _RANGE
    lanes = lax.iota(jnp.int32, 16)
    dump0 = _B + wid * 128

    pltpu.async_copy(keys_h, keysv, sem)
    pltpu.async_copy(hit_h, hitv, sem)
    pltpu.make_async_copy(keys_h, keysv, sem).wait()
    pltpu.make_async_copy(hit_h, hitv, sem).wait()


    def _scan(r, cnt):
        for l in range(8):
            kv = keysv[pl.ds(r * 128 + l * 16, 16)]
            rel = kv - base
            owned = (rel >= 0) & (rel < _RANGE)
            iv = r * 128 + l * 16 + lanes
            plsc.store_compressed(ilist.at[pl.ds(cnt, 16)], iv, mask=owned)
            plsc.store_compressed(klist.at[pl.ds(cnt, 16)], kv, mask=owned)
            pc = plsc.all_reduce_population_count(owned)
            cnt = cnt + jnp.max(pc)
        return cnt
    cnt = lax.fori_loop(0, _B // 128, _scan, jnp.int32(0))
    ilist[pl.ds(cnt, 16)] = jnp.full((16,), dump0, jnp.int32) + lanes
    klist[pl.ds(cnt, 16)] = jnp.full((16,), base, jnp.int32)
    nv = (cnt + 15) >> 4

    def _build(v, carry):
        iv = ilist[pl.ds(v * 16, 16)]
        valid = iv < _B
        rel = klist[pl.ds(v * 16, 16)] - base
        plsc.store_scatter(table, [rel], iv, mask=valid)
        return carry
    lax.fori_loop(0, nv, _build, 0)

    def _fix(_):
        def _sweep(v, bad):
            iv = ilist[pl.ds(v * 16, 16)]
            valid = iv < _B
            rel = klist[pl.ds(v * 16, 16)] - base
            w = plsc.load_gather(table, [rel], mask=valid)
            isbad = valid & (iv > w)
            plsc.store_scatter(table, [rel], iv, mask=isbad)
            nb = plsc.all_reduce_population_count(isbad)
            return bad + jnp.max(nb)
        return lax.fori_loop(0, nv, _sweep, jnp.int32(0))
    lax.while_loop(lambda bad: bad > 0, _fix, jnp.int32(1))

    def _part(v, carry):
        cnth, cntm = carry
        iv = ilist[pl.ds(v * 16, 16)]
        valid = iv < _B
        kv = klist[pl.ds(v * 16, 16)]
        rel = kv - base
        w = plsc.load_gather(table, [rel], mask=valid)
        hw = plsc.load_gather(hitv, [w], mask=valid)
        hb = valid & (hw > 0)
        mb = valid & jnp.logical_not(hb)
        plsc.store_compressed(ilist2.at[pl.ds(cnth, 16)], iv, mask=hb)
        plsc.store_compressed(klist2.at[pl.ds(cnth, 16)], kv, mask=hb)
        ph = plsc.all_reduce_population_count(hb)
        plsc.store_compressed(ilist.at[pl.ds(cntm, 16)], iv, mask=mb)
        plsc.store_compressed(klist.at[pl.ds(cntm, 16)],
                              jnp.where(valid, w, 0), mask=mb)
        pm = plsc.all_reduce_population_count(mb)
        return (cnth + jnp.max(ph), cntm + jnp.max(pm))
    cnth, cntm = lax.fori_loop(0, nv, _part, (jnp.int32(0), jnp.int32(0)))
    ilist2[pl.ds(cnth, 16)] = jnp.full((16,), dump0, jnp.int32) + lanes
    klist2[pl.ds(cnth, 16)] = jnp.full((16,), base, jnp.int32)
    def _scrub(t, carry):
        ilist[pl.ds(cntm + t * 16, 16)] = (
            jnp.full((16,), dump0, jnp.int32) + t * 16 + lanes)
        klist[pl.ds(cntm + t * 16, 16)] = jnp.zeros((16,), jnp.int32)
        return carry
    lax.fori_loop(0, 2, _scrub, 0)

    nvh = (cnth + 15) >> 4
    nvm = (cntm + 15) >> 4

    def _ado(v, carry):
        rs = klist2[pl.ds(v * 16, 16)]
        ivs = ilist2[pl.ds(v * 16, 16)]
        for l in range(16):
            pltpu.async_copy(mem_h.at[pl.ds(rs[l], 1)],
                             out_h.at[pl.ds(ivs[l], 1)], sem)
        return carry
    lax.fori_loop(0, nvh, _ado, 0)

    def _bdo(v, carry):
        ws = klist[pl.ds(v * 16, 16)]
        ivs = ilist[pl.ds(v * 16, 16)]
        for l in range(16):
            pltpu.async_copy(sm_h.at[pl.ds(ws[l], 1)],
                             out_h.at[pl.ds(ivs[l], 1)], sem)
        return carry
    lax.fori_loop(0, nvm, _bdo, 0)

    def _dr(j, carry):
        pltpu.make_async_copy(mem_h.at[pl.ds(0, 1)],
                              out_h.at[pl.ds(0, 1)], sem).wait()
        return carry
    lax.fori_loop(0, (nvh + nvm) * 16, _dr, 0)


_sparse = functools.partial(
    pl.kernel,
    out_type=jax.ShapeDtypeStruct((_B + _NW * 128, _D), jnp.float32),
    mesh=plsc.VectorSubcoreMesh(core_axis_name="c", subcore_axis_name="s"),
    scratch_types=[
        pltpu.VMEM((_B,), jnp.int32),
        pltpu.VMEM((_B,), jnp.int32),
        pltpu.VMEM((_TBL,), jnp.int32),
        pltpu.VMEM((_CAP,), jnp.int32),
        pltpu.VMEM((_CAP,), jnp.int32),
        pltpu.VMEM((_CAP,), jnp.int32),
        pltpu.VMEM((_CAP,), jnp.int32),
        pltpu.SemaphoreType.DMA,
    ],
    compiler_params=pltpu.CompilerParams(needs_layout_passes=False),
)(_sparse_body)


def kernel(mem, target, mask, keys, hit_flags):
    sm_miss = _stain_extract(target, mask)
    keys32 = keys.astype(jnp.int32)
    hit32 = hit_flags.astype(jnp.int32)
    out_pad = _sparse(mem, keys32, hit32, sm_miss)
    return out_pad[:_B]


# speedup vs baseline: 1.4587x; 1.0003x over previous
import functools

import jax
import jax.numpy as jnp
from jax import lax
from jax.experimental import pallas as pl
from jax.experimental.pallas import tpu as pltpu
from jax.experimental.pallas import tpu_sc as plsc

_M = 1000000
_B = 4096
_D = 6
_BB = 512
_NW = 32
_RANGE = _M // _NW
_TBL = _RANGE + 110
_CAP = _B + 128


def _stain_body(t_ref, m_ref, o_ref):
    t = t_ref[...]
    m = m_ref[...]
    denom = jnp.sum(m, axis=2) + 1e-6
    mean = jnp.sum(t * m, axis=2) / denom
    var = jnp.sum(((t - mean[:, :, None]) ** 2) * m, axis=2) / denom
    std = jnp.sqrt(var + 1e-6)
    o_ref[...] = jnp.concatenate(
        [mean[:, 0:1], std[:, 0:1], mean[:, 1:2], std[:, 1:2],
         mean[:, 2:3], std[:, 2:3]], axis=1)


def _stain_extract(target, mask):
    B, C, P = target.shape
    return pl.pallas_call(
        _stain_body,
        grid=(B // _BB,),
        in_specs=[
            pl.BlockSpec((_BB, C, P), lambda i: (i, 0, 0)),
            pl.BlockSpec((_BB, 1, P), lambda i: (i, 0, 0)),
        ],
        out_specs=pl.BlockSpec((_BB, _D), lambda i: (i, 0)),
        out_shape=jax.ShapeDtypeStruct((B, _D), jnp.float32),
    )(target, mask)


def _sparse_body(mem_h, keys_h, hit_h, sm_h, out_h,
                 keysv, hitv, table, ilist, klist, ilist2, klist2, sem):
    c = lax.axis_index("c")
    s = lax.axis_index("s")
    wid = s * 2 + c
    base = wid * _RANGE
    lanes = lax.iota(jnp.int32, 16)
    dump0 = _B + wid * 128

    pltpu.async_copy(keys_h, keysv, sem)
    pltpu.async_copy(hit_h, hitv, sem)
    pltpu.make_async_copy(keys_h, keysv, sem).wait()
    pltpu.make_async_copy(hit_h, hitv, sem).wait()


    def _scan(r, cnt):
        for l in range(8):
            kv = keysv[pl.ds(r * 128 + l * 16, 16)]
            rel = kv - base
            owned = (rel >= 0) & (rel < _RANGE)
            iv = r * 128 + l * 16 + lanes
            plsc.store_compressed(ilist.at[pl.ds(cnt, 16)], iv, mask=owned)
            plsc.store_compressed(klist.at[pl.ds(cnt, 16)], kv, mask=owned)
            pc = plsc.all_reduce_population_count(owned)
            cnt = cnt + jnp.max(pc)
        return cnt
    cnt = lax.fori_loop(0, _B // 128, _scan, jnp.int32(0))
    ilist[pl.ds(cnt, 16)] = jnp.full((16,), dump0, jnp.int32) + lanes
    klist[pl.ds(cnt, 16)] = jnp.full((16,), base, jnp.int32)
    nv = (cnt + 15) >> 4

    def _build(v, carry):
        iv = ilist[pl.ds(v * 16, 16)]
        valid = iv < _B
        rel = klist[pl.ds(v * 16, 16)] - base
        plsc.store_scatter(table, [rel], iv, mask=valid)
        return carry
    lax.fori_loop(0, nv, _build, 0)

    def _fix(_):
        def _sweep(v, bad):
            iv = ilist[pl.ds(v * 16, 16)]
            valid = iv < _B
            rel = klist[pl.ds(v * 16, 16)] - base
            w = plsc.load_gather(table, [rel], mask=valid)
            isbad = valid & (iv > w)
            plsc.store_scatter(table, [rel], iv, mask=isbad)
            nb = plsc.all_reduce_population_count(isbad)
            return bad + jnp.max(nb)
        return lax.fori_loop(0, nv, _sweep, jnp.int32(0))
    lax.while_loop(lambda bad: bad > 0, _fix, jnp.int32(1))

    def _part(v, carry):
        cnth, cntm = carry
        iv = ilist[pl.ds(v * 16, 16)]
        valid = iv < _B
        kv = klist[pl.ds(v * 16, 16)]
        rel = kv - base
        w = plsc.load_gather(table, [rel], mask=valid)
        hw = plsc.load_gather(hitv, [w], mask=valid)
        hb = valid & (hw > 0)
        mb = valid & jnp.logical_not(hb)
        plsc.store_compressed(ilist2.at[pl.ds(cnth, 16)], iv, mask=hb)
        plsc.store_compressed(klist2.at[pl.ds(cnth, 16)], kv, mask=hb)
        ph = plsc.all_reduce_population_count(hb)
        plsc.store_compressed(ilist.at[pl.ds(cntm, 16)], iv, mask=mb)
        plsc.store_compressed(klist.at[pl.ds(cntm, 16)],
                              jnp.where(valid, w, 0), mask=mb)
        pm = plsc.all_reduce_population_count(mb)
        return (cnth + jnp.max(ph), cntm + jnp.max(pm))
    cnth, cntm = lax.fori_loop(0, nv, _part, (jnp.int32(0), jnp.int32(0)))
    ilist2[pl.ds(cnth, 16)] = jnp.full((16,), dump0, jnp.int32) + lanes
    klist2[pl.ds(cnth, 16)] = jnp.full((16,), base, jnp.int32)
    def _scrub(t, carry):
        ilist[pl.ds(cntm + t * 16, 16)] = (
            jnp.full((16,), dump0, jnp.int32) + t * 16 + lanes)
        klist[pl.ds(cntm + t * 16, 16)] = jnp.zeros((16,), jnp.int32)
        return carry
    lax.fori_loop(0, 2, _scrub, 0)

    nvh = (cnth + 15) >> 4
    nvm = (cntm + 15) >> 4

    def _ado(v, carry):
        rs = klist2[pl.ds(v * 16, 16)]
        ivs = ilist2[pl.ds(v * 16, 16)]
        for l in range(16):
            pltpu.async_copy(mem_h.at[pl.ds(rs[l], 1)],
                             out_h.at[pl.ds(ivs[l], 1)], sem)
        return carry
    lax.fori_loop(0, nvh, _ado, 0)

    def _bdo(v, carry):
        ws = klist[pl.ds(v * 16, 16)]
        ivs = ilist[pl.ds(v * 16, 16)]
        for l in range(16):
            pltpu.async_copy(sm_h.at[pl.ds(ws[l], 1)],
                             out_h.at[pl.ds(ivs[l], 1)], sem)
        return carry
    lax.fori_loop(0, nvm, _bdo, 0)

    def _dr(j, carry):
        pltpu.make_async_copy(mem_h.at[pl.ds(0, 1)],
                              out_h.at[pl.ds(0, 1)], sem).wait()
        return carry
    lax.fori_loop(0, (nvh + nvm) * 16, _dr, 0)


_sparse = functools.partial(
    pl.kernel,
    out_type=jax.ShapeDtypeStruct((_B + _NW * 128, _D), jnp.float32),
    mesh=plsc.VectorSubcoreMesh(core_axis_name="c", subcore_axis_name="s"),
    scratch_types=[
        pltpu.VMEM((_B,), jnp.int32),
        pltpu.VMEM((_B,), jnp.int32),
        pltpu.VMEM((_TBL,), jnp.int32),
        pltpu.VMEM((_CAP,), jnp.int32),
        pltpu.VMEM((_CAP,), jnp.int32),
        pltpu.VMEM((_CAP,), jnp.int32),
        pltpu.VMEM((_CAP,), jnp.int32),
        pltpu.SemaphoreType.DMA,
    ],
    compiler_params=pltpu.CompilerParams(needs_layout_passes=False),
)(_sparse_body)


def kernel(mem, target, mask, keys, hit_flags):
    sm_miss = _stain_extract(target, mask)
    keys32 = keys.astype(jnp.int32)
    hit32 = hit_flags.astype(jnp.int32)
    out_pad = _sparse(mem, keys32, hit32, sm_miss)
    return out_pad[:_B]


# speedup vs baseline: 1.4663x; 1.0052x over previous
import functools

import jax
import jax.numpy as jnp
from jax import lax
from jax.experimental import pallas as pl
from jax.experimental.pallas import tpu as pltpu
from jax.experimental.pallas import tpu_sc as plsc

_M = 1000000
_B = 4096
_D = 6
_BB = 256
_NW = 32
_RANGE = _M // _NW
_TBL = _RANGE + 110
_CAP = _B + 128


def _stain_body(t_ref, m_ref, o_ref):
    t = t_ref[...]
    m = m_ref[...]
    denom = jnp.sum(m, axis=2) + 1e-6
    mean = jnp.sum(t * m, axis=2) / denom
    var = jnp.sum(((t - mean[:, :, None]) ** 2) * m, axis=2) / denom
    std = jnp.sqrt(var + 1e-6)
    o_ref[...] = jnp.concatenate(
        [mean[:, 0:1], std[:, 0:1], mean[:, 1:2], std[:, 1:2],
         mean[:, 2:3], std[:, 2:3]], axis=1)


def _stain_extract(target, mask):
    B, C, P = target.shape
    return pl.pallas_call(
        _stain_body,
        grid=(B // _BB,),
        in_specs=[
            pl.BlockSpec((_BB, C, P), lambda i: (i, 0, 0)),
            pl.BlockSpec((_BB, 1, P), lambda i: (i, 0, 0)),
        ],
        out_specs=pl.BlockSpec((_BB, _D), lambda i: (i, 0)),
        out_shape=jax.ShapeDtypeStruct((B, _D), jnp.float32),
    )(target, mask)


def _sparse_body(mem_h, keys_h, hit_h, sm_h, out_h,
                 keysv, hitv, table, ilist, klist, ilist2, klist2, sem):
    c = lax.axis_index("c")
    s = lax.axis_index("s")
    wid = s * 2 + c
    base = wid * _RANGE
    lanes = lax.iota(jnp.int32, 16)
    dump0 = _B + wid * 128

    pltpu.async_copy(keys_h, keysv, sem)
    pltpu.async_copy(hit_h, hitv, sem)
    pltpu.make_async_copy(keys_h, keysv, sem).wait()
    pltpu.make_async_copy(hit_h, hitv, sem).wait()


    def _scan(r, cnt):
        for l in range(8):
            kv = keysv[pl.ds(r * 128 + l * 16, 16)]
            rel = kv - base
            owned = (rel >= 0) & (rel < _RANGE)
            iv = r * 128 + l * 16 + lanes
            plsc.store_compressed(ilist.at[pl.ds(cnt, 16)], iv, mask=owned)
            plsc.store_compressed(klist.at[pl.ds(cnt, 16)], kv, mask=owned)
            pc = plsc.all_reduce_population_count(owned)
            cnt = cnt + jnp.max(pc)
        return cnt
    cnt = lax.fori_loop(0, _B // 128, _scan, jnp.int32(0))
    ilist[pl.ds(cnt, 16)] = jnp.full((16,), dump0, jnp.int32) + lanes
    klist[pl.ds(cnt, 16)] = jnp.full((16,), base, jnp.int32)
    nv = (cnt + 15) >> 4

    def _build(v, carry):
        iv = ilist[pl.ds(v * 16, 16)]
        valid = iv < _B
        rel = klist[pl.ds(v * 16, 16)] - base
        plsc.store_scatter(table, [rel], iv, mask=valid)
        return carry
    lax.fori_loop(0, nv, _build, 0)

    def _fix(_):
        def _sweep(v, bad):
            iv = ilist[pl.ds(v * 16, 16)]
            valid = iv < _B
            rel = klist[pl.ds(v * 16, 16)] - base
            w = plsc.load_gather(table, [rel], mask=valid)
            isbad = valid & (iv > w)
            plsc.store_scatter(table, [rel], iv, mask=isbad)
            nb = plsc.all_reduce_population_count(isbad)
            return bad + jnp.max(nb)
        return lax.fori_loop(0, nv, _sweep, jnp.int32(0))
    lax.while_loop(lambda bad: bad > 0, _fix, jnp.int32(1))

    def _part(v, carry):
        cnth, cntm = carry
        iv = ilist[pl.ds(v * 16, 16)]
        valid = iv < _B
        kv = klist[pl.ds(v * 16, 16)]
        rel = kv - base
        w = plsc.load_gather(table, [rel], mask=valid)
        hw = plsc.load_gather(hitv, [w], mask=valid)
        hb = valid & (hw > 0)
        mb = valid & jnp.logical_not(hb)
        plsc.store_compressed(ilist2.at[pl.ds(cnth, 16)], iv, mask=hb)
        plsc.store_compressed(klist2.at[pl.ds(cnth, 16)], kv, mask=hb)
        ph = plsc.all_reduce_population_count(hb)
        plsc.store_compressed(ilist.at[pl.ds(cntm, 16)], iv, mask=mb)
        plsc.store_compressed(klist.at[pl.ds(cntm, 16)],
                              jnp.where(valid, w, 0), mask=mb)
        pm = plsc.all_reduce_population_count(mb)
        return (cnth + jnp.max(ph), cntm + jnp.max(pm))
    cnth, cntm = lax.fori_loop(0, nv, _part, (jnp.int32(0), jnp.int32(0)))
    ilist2[pl.ds(cnth, 16)] = jnp.full((16,), dump0, jnp.int32) + lanes
    klist2[pl.ds(cnth, 16)] = jnp.full((16,), base, jnp.int32)
    def _scrub(t, carry):
        ilist[pl.ds(cntm + t * 16, 16)] = (
            jnp.full((16,), dump0, jnp.int32) + t * 16 + lanes)
        klist[pl.ds(cntm + t * 16, 16)] = jnp.zeros((16,), jnp.int32)
        return carry
    lax.fori_loop(0, 2, _scrub, 0)

    nvh = (cnth + 15) >> 4
    nvm = (cntm + 15) >> 4

    def _ado(v, carry):
        rs = klist2[pl.ds(v * 16, 16)]
        ivs = ilist2[pl.ds(v * 16, 16)]
        for l in range(16):
            pltpu.async_copy(mem_h.at[pl.ds(rs[l], 1)],
                             out_h.at[pl.ds(ivs[l], 1)], sem)
        return carry
    lax.fori_loop(0, nvh, _ado, 0)

    def _bdo(v, carry):
        ws = klist[pl.ds(v * 16, 16)]
        ivs = ilist[pl.ds(v * 16, 16)]
        for l in range(16):
            pltpu.async_copy(sm_h.at[pl.ds(ws[l], 1)],
                             out_h.at[pl.ds(ivs[l], 1)], sem)
        return carry
    lax.fori_loop(0, nvm, _bdo, 0)

    def _dr(j, carry):
        pltpu.make_async_copy(mem_h.at[pl.ds(0, 1)],
                              out_h.at[pl.ds(0, 1)], sem).wait()
        return carry
    lax.fori_loop(0, (nvh + nvm) * 16, _dr, 0)


_sparse = functools.partial(
    pl.kernel,
    out_type=jax.ShapeDtypeStruct((_B + _NW * 128, _D), jnp.float32),
    mesh=plsc.VectorSubcoreMesh(core_axis_name="c", subcore_axis_name="s"),
    scratch_types=[
        pltpu.VMEM((_B,), jnp.int32),
        pltpu.VMEM((_B,), jnp.int32),
        pltpu.VMEM((_TBL,), jnp.int32),
        pltpu.VMEM((_CAP,), jnp.int32),
        pltpu.VMEM((_CAP,), jnp.int32),
        pltpu.VMEM((_CAP,), jnp.int32),
        pltpu.VMEM((_CAP,), jnp.int32),
        pltpu.SemaphoreType.DMA,
    ],
    compiler_params=pltpu.CompilerParams(needs_layout_passes=False),
)(_sparse_body)


def kernel(mem, target, mask, keys, hit_flags):
    sm_miss = _stain_extract(target, mask)
    keys32 = keys.astype(jnp.int32)
    hit32 = hit_flags.astype(jnp.int32)
    out_pad = _sparse(mem, keys32, hit32, sm_miss)
    return out_pad[:_B]
